# Initial kernel scaffold; baseline (speedup 1.0000x reference)
import functools

import jax
import jax.numpy as jnp
from jax import lax
from jax.experimental import pallas as pl
from jax.experimental.pallas import tpu as pltpu
from jax.experimental.pallas import tpu_sc as plsc

N_IN = 16384
N_OUT = 16384
B = 64

NC = 2
NS = 16
NW = NC * NS

SUB = 128
S = 6
PER_W_SUBS = 66
NNZ_PAD = NW * PER_W_SUBS * SUB
N_BLOCKS = PER_W_SUBS // S

_ZCHUNK = 128
_ROWS_PER_TILE = N_OUT // NS


def _sc_scatter(xT, w2, rows2, cols2):
    mesh = plsc.VectorSubcoreMesh(core_axis_name="c", subcore_axis_name="s")

    @functools.partial(
        pl.kernel,
        mesh=mesh,
        out_type=jax.ShapeDtypeStruct((NC * N_OUT, B), jnp.float32),
        scratch_types=[
            pltpu.VMEM((S, SUB), jnp.int32),
            pltpu.VMEM((S, SUB), jnp.int32),
            pltpu.VMEM((S, SUB), jnp.float32),
            pltpu.VMEM((S, SUB, B), jnp.float32),
            pltpu.VMEM_SHARED((N_OUT, B), jnp.float32),
            pltpu.SemaphoreType.DMA,
        ],
    )
    def k(xT_hbm, w_hbm, rows_hbm, cols_hbm, out_hbm,
          cols_v, rows_v, w_v, g_v, acc, sem):
        c = lax.axis_index("c")
        s = lax.axis_index("s")
        wid = c * NS + s

        def zbody(i, _):
            for j4 in range(B // 16):
                g_v[0, i, pl.ds(j4 * 16, 16)] = jnp.zeros((16,), jnp.float32)
            return 0
        lax.fori_loop(0, _ZCHUNK, zbody, 0)
        zrow = s * _ROWS_PER_TILE
        for t in range(_ROWS_PER_TILE // _ZCHUNK):
            pltpu.sync_copy(g_v.at[0], acc.at[pl.ds(zrow + t * _ZCHUNK, _ZCHUNK)])
        plsc.subcore_barrier()

        def block_body(blk, _):
            rbase = wid * PER_W_SUBS + blk * S
            pltpu.sync_copy(cols_hbm.at[pl.ds(rbase, S)], cols_v)
            pltpu.sync_copy(rows_hbm.at[pl.ds(rbase, S)], rows_v)
            pltpu.sync_copy(w_hbm.at[pl.ds(rbase, S)], w_v)
            for j in range(S):
                pltpu.async_copy(xT_hbm.at[cols_v.at[j]], g_v.at[j], sem).wait()
            for j in range(S):
                def scale_body(kk, _, j=j):
                    idxj = jnp.full((16,), j, jnp.int32)
                    idxk = jnp.full((16,), kk, jnp.int32)
                    ws = plsc.load_gather(w_v, [idxj, idxk])
                    for j4 in range(B // 16):
                        sl = pl.ds(j4 * 16, 16)
                        g_v[j, kk, sl] = g_v[j, kk, sl] * ws
                    return 0
                lax.fori_loop(0, SUB, scale_body, 0)
            for j in range(S):
                pltpu.sync_copy(g_v.at[j], acc.at[rows_v.at[j]], add=True)
            return 0
        lax.fori_loop(0, N_BLOCKS, block_body, 0)

        plsc.subcore_barrier()
        orow = s * _ROWS_PER_TILE
        pltpu.sync_copy(acc.at[pl.ds(orow, _ROWS_PER_TILE)],
                        out_hbm.at[pl.ds(c * N_OUT + orow, _ROWS_PER_TILE)])

    return k(xT, w2, rows2, cols2)


def _combine(partials, bias_col):
    BLK = 1024

    def body(p0_ref, p1_ref, b_ref, o_ref):
        o_ref[...] = p0_ref[...] + p1_ref[...] + b_ref[...]

    return pl.pallas_call(
        body,
        grid=(N_OUT // BLK,),
        in_specs=[
            pl.BlockSpec((BLK, B), lambda i: (i, 0)),
            pl.BlockSpec((BLK, B), lambda i: (i + N_OUT // BLK, 0)),
            pl.BlockSpec((BLK, 1), lambda i: (i, 0)),
        ],
        out_specs=pl.BlockSpec((BLK, B), lambda i: (i, 0)),
        out_shape=jax.ShapeDtypeStruct((N_OUT, B), jnp.float32),
    )(partials, partials, bias_col)


def kernel(x, sparse_weight, bias, rows, cols):
    nnz = sparse_weight.shape[0]
    npad = NNZ_PAD - nnz
    xT = x.T
    w2 = jnp.pad(sparse_weight, (0, npad)).reshape(-1, SUB)
    rows2 = jnp.pad(rows, (0, npad)).reshape(-1, SUB)
    cols2 = jnp.pad(cols, (0, npad)).reshape(-1, SUB)
    partials = _sc_scatter(xT, w2, rows2, cols2)
    outT = _combine(partials, bias.reshape(-1, 1))
    return outT.T


# speedup vs baseline: 3.2464x; 3.2464x over previous
import functools

import jax
import jax.numpy as jnp
from jax import lax
from jax.experimental import pallas as pl
from jax.experimental.pallas import tpu as pltpu
from jax.experimental.pallas import tpu_sc as plsc

N_IN = 16384
N_OUT = 16384
B = 64

NC = 2
NS = 16
NW = NC * NS

SUB = 128
S = 8
PER_W_SUBS = 72
NNZ_PAD = NW * PER_W_SUBS * SUB
N_BLOCKS = PER_W_SUBS // S

_ZCHUNK = 128
_ROWS_PER_TILE = N_OUT // NS

Q = 2
N_CHUNKS = PER_W_SUBS // Q
N_PAIRS = N_CHUNKS // 2


def _sc_scatter(xT, w2, rows2, cols2):
    mesh = plsc.VectorSubcoreMesh(core_axis_name="c", subcore_axis_name="s")

    @functools.partial(
        pl.kernel,
        mesh=mesh,
        out_type=jax.ShapeDtypeStruct((NC * N_OUT, B), jnp.float32),
        scratch_types=[
            pltpu.VMEM((PER_W_SUBS * SUB,), jnp.int32),
            pltpu.VMEM((PER_W_SUBS * SUB,), jnp.int32),
            pltpu.VMEM((PER_W_SUBS * SUB,), jnp.float32),
            pltpu.VMEM((Q * SUB, B), jnp.float32),
            pltpu.VMEM((Q * SUB, B), jnp.float32),
            pltpu.VMEM_SHARED((N_OUT, B), jnp.float32),
            pltpu.SemaphoreType.DMA,
            pltpu.SemaphoreType.DMA,
            pltpu.SemaphoreType.DMA,
            pltpu.SemaphoreType.DMA,
        ],
        compiler_params=pltpu.CompilerParams(use_tc_tiling_on_sc=False),
    )
    def k(xT_hbm, w_hbm, rows_hbm, cols_hbm, out_hbm,
          cols_v, rows_v, w_v, g_a, g_b, acc, sga, sgb, ssa, ssb):
        c = lax.axis_index("c")
        s = lax.axis_index("s")
        wid = c * NS + s

        def zbody(i, _):
            for j4 in range(B // 16):
                g_a[i, pl.ds(j4 * 16, 16)] = jnp.zeros((16,), jnp.float32)
            return 0
        lax.fori_loop(0, _ZCHUNK, zbody, 0)
        zrow = s * _ROWS_PER_TILE
        for t in range(_ROWS_PER_TILE // _ZCHUNK):
            pltpu.sync_copy(g_a.at[pl.ds(0, _ZCHUNK)],
                            acc.at[pl.ds(zrow + t * _ZCHUNK, _ZCHUNK)])

        rbase = wid * PER_W_SUBS * SUB
        pltpu.sync_copy(cols_hbm.at[pl.ds(rbase, PER_W_SUBS * SUB)], cols_v)
        pltpu.sync_copy(rows_hbm.at[pl.ds(rbase, PER_W_SUBS * SUB)], rows_v)
        pltpu.sync_copy(w_hbm.at[pl.ds(rbase, PER_W_SUBS * SUB)], w_v)
        plsc.subcore_barrier()

        def gather_issue(q, buf, sem):
            pltpu.async_copy(xT_hbm.at[cols_v.at[pl.ds(Q * SUB * q, Q * SUB)]], buf, sem)

        def gather_wait(buf, sem):
            pltpu.make_async_copy(xT_hbm.at[cols_v.at[pl.ds(0, Q * SUB)]], buf,
                                  sem).wait()

        def scatter_issue(q, buf, sem):
            pltpu.async_copy(buf, acc.at[rows_v.at[pl.ds(Q * SUB * q, Q * SUB)]], sem,
                             add=True)

        def scatter_wait(buf, sem):
            pltpu.make_async_copy(buf, acc.at[rows_v.at[pl.ds(0, Q * SUB)]],
                                  sem).wait()

        def scale(q, buf):
            def scale_body(gi, _):
                kb = gi * 16
                w16 = w_v[pl.ds(q * Q * SUB + kb, 16)]
                for i in range(16):
                    ws = lax.gather(
                        w16, jnp.full((16, 1), i, jnp.int32),
                        lax.GatherDimensionNumbers(
                            offset_dims=(), collapsed_slice_dims=(0,),
                            start_index_map=(0,)),
                        slice_sizes=(1,),
                        mode=lax.GatherScatterMode.PROMISE_IN_BOUNDS)
                    for j4 in range(B // 16):
                        sl = pl.ds(j4 * 16, 16)
                        buf[kb + i, sl] = buf[kb + i, sl] * ws
                return 0
            lax.fori_loop(0, Q * SUB // 16, scale_body, 0)

        gather_issue(0, g_a, sga)

        def pair_body(i, _):
            q0 = 2 * i
            gather_wait(g_a, sga)

            @pl.when(i > 0)
            def _():
                scatter_wait(g_b, ssb)
            gather_issue(q0 + 1, g_b, sgb)
            scale(q0, g_a)
            scatter_issue(q0, g_a, ssa)
            gather_wait(g_b, sgb)
            scatter_wait(g_a, ssa)

            @pl.when(i < N_PAIRS - 1)
            def _():
                gather_issue(q0 + 2, g_a, sga)
            scale(q0 + 1, g_b)
            scatter_issue(q0 + 1, g_b, ssb)
            return 0
        lax.fori_loop(0, N_PAIRS, pair_body, 0)
        scatter_wait(g_b, ssb)

        plsc.subcore_barrier()
        orow = s * _ROWS_PER_TILE
        pltpu.sync_copy(acc.at[pl.ds(orow, _ROWS_PER_TILE)],
                        out_hbm.at[pl.ds(c * N_OUT + orow, _ROWS_PER_TILE)])

    return k(xT, w2, rows2, cols2)


def _combine(partials, bias_col):
    BLK = 1024

    def body(p0_ref, p1_ref, b_ref, o_ref):
        o_ref[...] = p0_ref[...] + p1_ref[...] + b_ref[...]

    return pl.pallas_call(
        body,
        grid=(N_OUT // BLK,),
        in_specs=[
            pl.BlockSpec((BLK, B), lambda i: (i, 0)),
            pl.BlockSpec((BLK, B), lambda i: (i + N_OUT // BLK, 0)),
            pl.BlockSpec((BLK, 1), lambda i: (i, 0)),
        ],
        out_specs=pl.BlockSpec((BLK, B), lambda i: (i, 0)),
        out_shape=jax.ShapeDtypeStruct((N_OUT, B), jnp.float32),
    )(partials, partials, bias_col)


def kernel(x, sparse_weight, bias, rows, cols):
    nnz = sparse_weight.shape[0]
    npad = NNZ_PAD - nnz
    xT = x.T
    w2 = jnp.pad(sparse_weight, (0, npad))
    rows2 = jnp.pad(rows, (0, npad))
    cols2 = jnp.pad(cols, (0, npad))
    partials = _sc_scatter(xT, w2, rows2, cols2)
    outT = _combine(partials, bias.reshape(-1, 1))
    return outT.T
